# baseline (speedup 1.0000x reference)
import functools

import jax
import jax.numpy as jnp
from jax import lax
from jax.experimental import pallas as pl
from jax.experimental.pallas import tpu as pltpu
from jax.experimental.pallas import tpu_sc as plsc

B = 10000
D = 128
DA = 144
B_PAD = 10240
NC, NS = 2, 16
NW = NC * NS
CHUNK = 64
E_TOT = 320000 + B
CH_PER_W = -(-E_TOT // (NW * CHUNK))
IDX_GRP = 18
TOTAL_CH = CH_PER_W * NW
E_PAD = TOTAL_CH * CHUNK
B_ACC = 10240
ROWS_PER_TILE = B_ACC // NS
ZROWS = CHUNK
R_BLK = 1024
F_BLK = 1000


def _prep_body(ent_ref, waug_ref, baug_ref, wv1_ref, tab_ref, al_ref):
    i = pl.program_id(0)
    x = ent_ref[...]
    y = jnp.dot(x, waug_ref[...], preferred_element_type=jnp.float32) + baug_ref[...]
    rowid = i * R_BLK + lax.broadcasted_iota(jnp.int32, (R_BLK, 1), 0)
    colid = lax.broadcasted_iota(jnp.int32, (1, DA), 1)
    sent = jnp.where(colid == 129, jnp.float32(1e9), jnp.float32(0.0))
    tab_ref[...] = jnp.where(rowid >= B, sent, y)
    al_ref[...] = jnp.sum(x * wv1_ref[...], axis=1).reshape(1, 1, R_BLK)


def _fin_body(p_ref, a_ref, o_ref):
    p = p_ref[...]
    num = p[0, :, :D] + p[1, :, :D]
    den = p[0, :, D:D + 1] + p[1, :, D:D + 1]
    o = num / den
    a = a_ref[0, 0]
    o_ref[...] = jnp.where(o >= 0, o, a * o)


def _edge_body(tab_hbm, al_hbm, row_hbm, col_hbm, out_hbm,
               row_v, col_v, al_v, rows0_v, rows1_v, acc_sh,
               semg0, semg1, sems0, sems1):
    c = lax.axis_index("c")
    s = lax.axis_index("s")
    w = c * NS + s
    rows = (rows0_v, rows1_v)
    semg = (semg0, semg1)
    sems = (sems0, sems1)

    def zrow(j, _):
        for g in range(DA // 16):
            rows0_v[j, pl.ds(g * 16, 16)] = jnp.zeros((16,), jnp.float32)
        return 0
    lax.fori_loop(0, ZROWS, zrow, 0)
    base = s * ROWS_PER_TILE
    for t in range(ROWS_PER_TILE // ZROWS):
        pltpu.sync_copy(rows0_v, acc_sh.at[pl.ds(base + t * ZROWS, ZROWS)])

    pltpu.sync_copy(al_hbm, al_v)

    plsc.subcore_barrier()

    def gather_start(k, b):
        pltpu.async_copy(tab_hbm.at[col_v.at[k]], rows[b], semg[b])

    def gather_wait(k, b):
        pltpu.make_async_copy(tab_hbm.at[col_v.at[k]], rows[b], semg[b]).wait()

    def scatter_start(k, b):
        pltpu.async_copy(rows[b], acc_sh.at[row_v.at[k]], sems[b], add=True)

    def scatter_wait(k, b):
        pltpu.make_async_copy(rows[b], acc_sh.at[row_v.at[k]], sems[b]).wait()

    def compute(k, b):
        rbuf = rows[b]
        for g in range(CHUNK // 16):
            rv = row_v[k, pl.ds(g * 16, 16)]
            alv = plsc.load_gather(al_v, [rv])
            ei = lax.iota(jnp.int32, 16) + g * 16
            arv = plsc.load_gather(rbuf, [ei, jnp.full((16,), D + 1, jnp.int32)])
            sc = alv + arv
            lk = jnp.where(sc >= 0, sc, jnp.float32(0.2) * sc)
            attv = jnp.exp(-lk)
            for e2 in range(16):
                e = g * 16 + e2
                a = jnp.broadcast_to(attv[e2], (16,))
                for j in range(DA // 16):
                    rbuf[e, pl.ds(j * 16, 16)] = rbuf[e, pl.ds(j * 16, 16)] * a

    def grp_body(gidx, _):
        pltpu.sync_copy(row_hbm.at[w, pl.ds(gidx * IDX_GRP, IDX_GRP)], row_v)
        pltpu.sync_copy(col_hbm.at[w, pl.ds(gidx * IDX_GRP, IDX_GRP)], col_v)

        gather_start(0, 0)

        def pair_body(p, _):
            k = p * 2

            @pl.when(p > 0)
            def _():
                scatter_wait(k - 1, 1)
            gather_start(k + 1, 1)

            gather_wait(k, 0)
            compute(k, 0)
            scatter_start(k, 0)

            gather_wait(k + 1, 1)
            compute(k + 1, 1)
            scatter_start(k + 1, 1)

            scatter_wait(k, 0)

            @pl.when(k + 2 < IDX_GRP)
            def _():
                gather_start(k + 2, 0)
            return 0
        lax.fori_loop(0, IDX_GRP // 2, pair_body, 0)
        scatter_wait(IDX_GRP - 1, 1)
        return 0
    lax.fori_loop(0, CH_PER_W // IDX_GRP, grp_body, 0)

    plsc.subcore_barrier()
    pltpu.sync_copy(acc_sh.at[pl.ds(base, ROWS_PER_TILE)],
                    out_hbm.at[c, pl.ds(base, ROWS_PER_TILE)])


_edge_kernel = functools.partial(
    pl.kernel,
    _edge_body,
    out_type=jax.ShapeDtypeStruct((NC, B_ACC, DA), jnp.float32),
    mesh=plsc.VectorSubcoreMesh(core_axis_name="c", subcore_axis_name="s"),
    compiler_params=pltpu.CompilerParams(
        needs_layout_passes=False, use_tc_tiling_on_sc=False),
    scratch_types=[
        pltpu.VMEM((IDX_GRP, CHUNK), jnp.int32),
        pltpu.VMEM((IDX_GRP, CHUNK), jnp.int32),
        pltpu.VMEM((B_PAD,), jnp.float32),
        pltpu.VMEM((CHUNK, DA), jnp.float32),
        pltpu.VMEM((CHUNK, DA), jnp.float32),
        pltpu.VMEM_SHARED((B_ACC, DA), jnp.float32),
        pltpu.SemaphoreType.DMA,
        pltpu.SemaphoreType.DMA,
        pltpu.SemaphoreType.DMA,
        pltpu.SemaphoreType.DMA,
    ],
)()


def kernel(batch_ids, batch_adj_arr, ent_embed, feature_dropout, W, b, w_atten_r, prelu_a):
    w1 = w_atten_r[:D, 0]
    w2 = w_atten_r[D:, 0]
    W_aug = (jnp.zeros((D, DA), jnp.float32)
             .at[:, :D].set(W)
             .at[:, D + 1].set(W @ w2))
    b_aug = (jnp.zeros((DA,), jnp.float32)
             .at[:D].set(b)
             .at[D].set(1.0)
             .at[D + 1].set(jnp.dot(b, w2)))
    wv1 = W @ w1
    ent_pad = jnp.zeros((B_PAD, D), jnp.float32).at[:B].set(ent_embed)

    tab, al2 = pl.pallas_call(
        _prep_body,
        out_shape=(
            jax.ShapeDtypeStruct((B_PAD, DA), jnp.float32),
            jax.ShapeDtypeStruct((B_PAD // R_BLK, 1, R_BLK), jnp.float32),
        ),
        grid=(B_PAD // R_BLK,),
        in_specs=[
            pl.BlockSpec((R_BLK, D), lambda i: (i, 0)),
            pl.BlockSpec((D, DA), lambda i: (0, 0)),
            pl.BlockSpec((1, DA), lambda i: (0, 0)),
            pl.BlockSpec((1, D), lambda i: (0, 0)),
        ],
        out_specs=(
            pl.BlockSpec((R_BLK, DA), lambda i: (i, 0)),
            pl.BlockSpec((1, 1, R_BLK), lambda i: (i, 0, 0)),
        ),
    )(ent_pad, W_aug, b_aug[None, :], wv1[None, :])
    al = al2.reshape(B_PAD) + jnp.dot(b, w1)

    row = jnp.concatenate([batch_adj_arr[0], jnp.arange(B, dtype=jnp.int32)])
    col = jnp.concatenate([batch_adj_arr[1], batch_ids.astype(jnp.int32)])
    rowp = jnp.zeros((E_PAD,), jnp.int32).at[:E_TOT].set(row).reshape(NW, CH_PER_W, CHUNK)
    colp = jnp.full((E_PAD,), B, jnp.int32).at[:E_TOT].set(col).reshape(NW, CH_PER_W, CHUNK)

    parts = _edge_kernel(tab, al, rowp, colp)

    return pl.pallas_call(
        _fin_body,
        out_shape=jax.ShapeDtypeStruct((B, D), jnp.float32),
        grid=(B // F_BLK,),
        in_specs=[
            pl.BlockSpec((NC, F_BLK, DA), lambda i: (0, i, 0)),
            pl.BlockSpec((1, 1), lambda i: (0, 0)),
        ],
        out_specs=pl.BlockSpec((F_BLK, D), lambda i: (i, 0)),
    )(parts, jnp.float32(prelu_a).reshape(1, 1))


# speedup vs baseline: 1.1034x; 1.1034x over previous
import functools

import jax
import jax.numpy as jnp
from jax import lax
from jax.experimental import pallas as pl
from jax.experimental.pallas import tpu as pltpu
from jax.experimental.pallas import tpu_sc as plsc

B = 10000
D = 128
DH = 64
DAH = 80
B_PAD = 10240
NC, NS = 2, 16
CHUNK = 64
E_TOT = 320000 + B
CH_PER_S = -(-E_TOT // (NS * CHUNK))
CH_PER_S = 324
E_PAD = CH_PER_S * NS * CHUNK
IDX_GRP = 108
NGRP = CH_PER_S // IDX_GRP
DEPTH = 6
LAG = 5
B_ACC = 10240
ROWS_PER_TILE = B_ACC // NS
ZROWS = CHUNK
R_BLK = 1024
F_BLK = 1000


def _prep_body(ent_ref, waug_ref, baug_ref, wv1_ref, tab_ref, al_ref):
    i = pl.program_id(0)
    x = ent_ref[...]
    y = jnp.dot(x, waug_ref[0], preferred_element_type=jnp.float32) + baug_ref[0]
    rowid = i * R_BLK + lax.broadcasted_iota(jnp.int32, (R_BLK, 1), 0)
    colid = lax.broadcasted_iota(jnp.int32, (1, DAH), 1)
    sent = jnp.where(colid == DH + 1, jnp.float32(1e9), jnp.float32(0.0))
    tab_ref[...] = jnp.where(rowid >= B, sent, y)[None]
    al_ref[...] = jnp.sum(x * wv1_ref[...], axis=1).reshape(1, 1, R_BLK)


def _fin_body(p_ref, a_ref, o_ref):
    p = p_ref[...]
    num = jnp.concatenate([p[0, :, :DH], p[1, :, :DH]], axis=1)
    den = p[0, :, DH:DH + 1]
    o = num / den
    a = a_ref[0, 0]
    o_ref[...] = jnp.where(o >= 0, o, a * o)


def _edge_body(tab_hbm, al_hbm, row_hbm, col_hbm, out_hbm,
               row_v, col_v, al_v, rows_v, att_v, acc_sh, semg, sems, semi):
    c = lax.axis_index("c")
    s = lax.axis_index("s")

    def zrow(j, _):
        for g in range(DAH // 16):
            rows_v[0, j, pl.ds(g * 16, 16)] = jnp.zeros((16,), jnp.float32)
        return 0
    lax.fori_loop(0, ZROWS, zrow, 0)
    base = s * ROWS_PER_TILE
    for t in range(ROWS_PER_TILE // ZROWS):
        pltpu.sync_copy(rows_v.at[0], acc_sh.at[pl.ds(base + t * ZROWS, ZROWS)])

    pltpu.sync_copy(al_hbm, al_v)
    pltpu.sync_copy(row_hbm.at[s, pl.ds(0, IDX_GRP)], row_v.at[0])
    pltpu.sync_copy(col_hbm.at[c, s, pl.ds(0, IDX_GRP)], col_v.at[0])

    plsc.subcore_barrier()

    def gather_start(slot, k, b):
        pltpu.async_copy(tab_hbm.at[col_v.at[slot].at[k]], rows_v.at[b],
                         semg.at[b])

    def gather_wait(slot, k, b):
        pltpu.make_async_copy(tab_hbm.at[col_v.at[slot].at[k]], rows_v.at[b],
                              semg.at[b]).wait()

    def scatter_start(slot, k, b):
        pltpu.async_copy(rows_v.at[b], acc_sh.at[row_v.at[slot].at[k]],
                         sems.at[b], add=True)

    def scatter_wait(slot, k, b):
        pltpu.make_async_copy(rows_v.at[b], acc_sh.at[row_v.at[slot].at[k]],
                              sems.at[b]).wait()

    def compute(slot, k, b):
        rbuf = rows_v.at[b]
        for g in range(CHUNK // 16):
            rv = row_v[slot, k, pl.ds(g * 16, 16)]
            alv = plsc.load_gather(al_v, [rv])
            ei = lax.iota(jnp.int32, 16) + g * 16
            arv = plsc.load_gather(rbuf, [ei, jnp.full((16,), DH + 1, jnp.int32)])
            sc = alv + arv
            lk = jnp.where(sc >= 0, sc, jnp.float32(0.2) * sc)
            att_v[pl.ds(g * 16, 16)] = jnp.exp(-lk)

        @plsc.parallel_loop(0, CHUNK, unroll=4)
        def scale_body(e):
            a = plsc.load_gather(att_v, [jnp.broadcast_to(e, (16,))])
            for j in range(DAH // 16):
                rbuf[e, pl.ds(j * 16, 16)] = rbuf[e, pl.ds(j * 16, 16)] * a

    N_QUAD = -(-(IDX_GRP + DEPTH) // DEPTH)
    for grp in range(NGRP):
        slot = grp % 2
        nslot = (grp + 1) % 2
        if grp > 0:
            pltpu.make_async_copy(
                row_hbm.at[s, pl.ds(grp * IDX_GRP, IDX_GRP)], row_v.at[slot],
                semi.at[slot]).wait()
            pltpu.make_async_copy(
                col_hbm.at[c, s, pl.ds(grp * IDX_GRP, IDX_GRP)], col_v.at[slot],
                semi.at[slot]).wait()
        if grp + 1 < NGRP:
            pltpu.async_copy(
                row_hbm.at[s, pl.ds((grp + 1) * IDX_GRP, IDX_GRP)],
                row_v.at[nslot], semi.at[nslot])
            pltpu.async_copy(
                col_hbm.at[c, s, pl.ds((grp + 1) * IDX_GRP, IDX_GRP)],
                col_v.at[nslot], semi.at[nslot])

        def quad(p, _, slot=slot):
            k0 = p * DEPTH
            for i in range(DEPTH):
                k = k0 + i
                kc = k - LAG
                bc = (i - LAG) % DEPTH

                @pl.when((k >= DEPTH) & (k < IDX_GRP + DEPTH))
                def _():
                    scatter_wait(slot, k - DEPTH, i)

                @pl.when(k < IDX_GRP)
                def _():
                    gather_start(slot, k, i)

                @pl.when((kc >= 0) & (kc < IDX_GRP))
                def _():
                    gather_wait(slot, kc, bc)
                    compute(slot, kc, bc)
                    scatter_start(slot, kc, bc)
            return 0
        lax.fori_loop(0, N_QUAD, quad, 0)

    plsc.subcore_barrier()
    pltpu.sync_copy(acc_sh.at[pl.ds(base, ROWS_PER_TILE)],
                    out_hbm.at[c, pl.ds(base, ROWS_PER_TILE)])


_edge_kernel = functools.partial(
    pl.kernel,
    _edge_body,
    out_type=jax.ShapeDtypeStruct((NC, B_ACC, DAH), jnp.float32),
    mesh=plsc.VectorSubcoreMesh(core_axis_name="c", subcore_axis_name="s"),
    compiler_params=pltpu.CompilerParams(
        needs_layout_passes=False, use_tc_tiling_on_sc=False),
    scratch_types=[
        pltpu.VMEM((2, IDX_GRP, CHUNK), jnp.int32),
        pltpu.VMEM((2, IDX_GRP, CHUNK), jnp.int32),
        pltpu.VMEM((B_PAD,), jnp.float32),
        pltpu.VMEM((DEPTH, CHUNK, DAH), jnp.float32),
        pltpu.VMEM((CHUNK,), jnp.float32),
        pltpu.VMEM_SHARED((B_ACC, DAH), jnp.float32),
        pltpu.SemaphoreType.DMA((DEPTH,)),
        pltpu.SemaphoreType.DMA((DEPTH,)),
        pltpu.SemaphoreType.DMA((2,)),
    ],
)()


def kernel(batch_ids, batch_adj_arr, ent_embed, feature_dropout, W, b, w_atten_r, prelu_a):
    w1 = w_atten_r[:D, 0]
    w2 = w_atten_r[D:, 0]
    wv2 = W @ w2
    W_aug = (jnp.zeros((NC, D, DAH), jnp.float32)
             .at[0, :, :DH].set(W[:, :DH])
             .at[1, :, :DH].set(W[:, DH:])
             .at[:, :, DH + 1].set(wv2))
    bv2 = jnp.dot(b, w2)
    b_aug = (jnp.zeros((NC, 1, DAH), jnp.float32)
             .at[0, 0, :DH].set(b[:DH])
             .at[1, 0, :DH].set(b[DH:])
             .at[:, 0, DH].set(1.0)
             .at[:, 0, DH + 1].set(bv2))
    wv1 = W @ w1
    ent_pad = jnp.zeros((B_PAD, D), jnp.float32).at[:B].set(ent_embed)

    tab, al2 = pl.pallas_call(
        _prep_body,
        out_shape=(
            jax.ShapeDtypeStruct((NC, B_PAD, DAH), jnp.float32),
            jax.ShapeDtypeStruct((B_PAD // R_BLK, 1, R_BLK), jnp.float32),
        ),
        grid=(B_PAD // R_BLK, NC),
        in_specs=[
            pl.BlockSpec((R_BLK, D), lambda i, c: (i, 0)),
            pl.BlockSpec((1, D, DAH), lambda i, c: (c, 0, 0)),
            pl.BlockSpec((1, 1, DAH), lambda i, c: (c, 0, 0)),
            pl.BlockSpec((1, D), lambda i, c: (0, 0)),
        ],
        out_specs=(
            pl.BlockSpec((1, R_BLK, DAH), lambda i, c: (c, i, 0)),
            pl.BlockSpec((1, 1, R_BLK), lambda i, c: (i, 0, 0)),
        ),
    )(ent_pad, W_aug, b_aug, wv1[None, :])
    tab_flat = tab.reshape(NC * B_PAD, DAH)
    al = al2.reshape(B_PAD) + jnp.dot(b, w1)

    row = jnp.concatenate([batch_adj_arr[0], jnp.arange(B, dtype=jnp.int32)])
    col = jnp.concatenate([batch_adj_arr[1], batch_ids.astype(jnp.int32)])
    rowp = jnp.zeros((E_PAD,), jnp.int32).at[:E_TOT].set(row).reshape(NS, CH_PER_S, CHUNK)
    colp = jnp.full((E_PAD,), B, jnp.int32).at[:E_TOT].set(col).reshape(NS, CH_PER_S, CHUNK)
    colp2 = jnp.stack([colp, colp + B_PAD])

    parts = _edge_kernel(tab_flat, al, rowp, colp2)

    return pl.pallas_call(
        _fin_body,
        out_shape=jax.ShapeDtypeStruct((B, D), jnp.float32),
        grid=(B // F_BLK,),
        in_specs=[
            pl.BlockSpec((NC, F_BLK, DAH), lambda i: (0, i, 0)),
            pl.BlockSpec((1, 1), lambda i: (0, 0)),
        ],
        out_specs=pl.BlockSpec((F_BLK, D), lambda i: (i, 0)),
    )(parts, jnp.float32(prelu_a).reshape(1, 1))


# speedup vs baseline: 1.1471x; 1.0396x over previous
import functools

import jax
import jax.numpy as jnp
from jax import lax
from jax.experimental import pallas as pl
from jax.experimental.pallas import tpu as pltpu
from jax.experimental.pallas import tpu_sc as plsc

B = 10000
D = 128
DH = 64
B_PAD = 10240
NC, NS = 2, 16
CHUNK = 64
E_TOT = 320000 + B
CH_PER_S = 324
E_PAD = CH_PER_S * NS * CHUNK
IDX_GRP = 54
NGRP = CH_PER_S // IDX_GRP
DEPTH = 6
LAG = 5
B_ACC = 10240
ROWS_PER_TILE = B_ACC // NS
ZROWS = CHUNK
R_BLK = 1024
F_BLK = 1000


def _prep_body(ent_ref, waug_ref, baug_ref, wv1_ref, wv2_ref,
               tab_ref, al_ref, ar_ref):
    x = ent_ref[...]
    y = jnp.dot(x, waug_ref[0], preferred_element_type=jnp.float32) + baug_ref[0]
    tab_ref[...] = y[None]
    al_ref[...] = jnp.sum(x * wv1_ref[...], axis=1).reshape(1, 1, R_BLK)
    ar_ref[...] = jnp.sum(x * wv2_ref[...], axis=1).reshape(1, 1, R_BLK)


def _fin_body(p_ref, r_ref, a_ref, o_ref):
    p = p_ref[...]
    num = jnp.concatenate([p[0, :, :DH], p[1, :, :DH]], axis=1)
    den = r_ref[0, :, 0:1]
    o = num / den
    a = a_ref[0, 0]
    o_ref[...] = jnp.where(o >= 0, o, a * o)


def _edge_body(tab_hbm, al_hbm, ar_hbm, row_hbm, col_hbm, out_hbm, outr_hbm,
               row_v, col_v, al_v, ar_v, rows_v, attb_v, att_v,
               acc_sh, accr_sh, semg, sems, semr, semi):
    c = lax.axis_index("c")
    s = lax.axis_index("s")

    def zrow(j, _):
        for g in range(DH // 16):
            rows_v[0, j, pl.ds(g * 16, 16)] = jnp.zeros((16,), jnp.float32)
        attb_v[0, j, pl.ds(0, 16)] = jnp.zeros((16,), jnp.float32)
        return 0
    lax.fori_loop(0, ZROWS, zrow, 0)
    base = s * ROWS_PER_TILE
    for t in range(ROWS_PER_TILE // ZROWS):
        pltpu.sync_copy(rows_v.at[0], acc_sh.at[pl.ds(base + t * ZROWS, ZROWS)])
        pltpu.sync_copy(attb_v.at[0], accr_sh.at[pl.ds(base + t * ZROWS, ZROWS)])

    pltpu.sync_copy(al_hbm, al_v)
    pltpu.sync_copy(ar_hbm, ar_v)
    pltpu.sync_copy(row_hbm.at[s, pl.ds(0, IDX_GRP)], row_v.at[0])
    pltpu.sync_copy(col_hbm.at[c, s, pl.ds(0, IDX_GRP)], col_v.at[0])

    plsc.subcore_barrier()

    def gather_start(slot, k, b):
        pltpu.async_copy(tab_hbm.at[col_v.at[slot].at[k]], rows_v.at[b],
                         semg.at[b])

    def gather_wait(slot, k, b):
        pltpu.make_async_copy(tab_hbm.at[col_v.at[slot].at[k]], rows_v.at[b],
                              semg.at[b]).wait()

    def scatter_start(slot, k, b):
        pltpu.async_copy(rows_v.at[b], acc_sh.at[row_v.at[slot].at[k]],
                         sems.at[b], add=True)
        pltpu.async_copy(attb_v.at[b], accr_sh.at[row_v.at[slot].at[k]],
                         semr.at[b], add=True)

    def scatter_wait(slot, k, b):
        pltpu.make_async_copy(rows_v.at[b], acc_sh.at[row_v.at[slot].at[k]],
                              sems.at[b]).wait()
        pltpu.make_async_copy(attb_v.at[b], accr_sh.at[row_v.at[slot].at[k]],
                              semr.at[b]).wait()

    def compute(slot, k, b):
        rbuf = rows_v.at[b]
        abuf = attb_v.at[b]
        cb = jnp.broadcast_to(c * B_PAD, (16,))
        for g in range(CHUNK // 16):
            rv = row_v[slot, k, pl.ds(g * 16, 16)]
            alv = plsc.load_gather(al_v, [rv])
            cv = col_v[slot, k, pl.ds(g * 16, 16)] - cb
            arv = plsc.load_gather(ar_v, [cv])
            sc = alv + arv
            lk = jnp.where(sc >= 0, sc, jnp.float32(0.2) * sc)
            att_v[pl.ds(g * 16, 16)] = jnp.exp(-lk)

        @plsc.parallel_loop(0, CHUNK, unroll=4)
        def scale_body(e):
            a = plsc.load_gather(att_v, [jnp.broadcast_to(e, (16,))])
            abuf[e, pl.ds(0, 16)] = a
            for j in range(DH // 16):
                rbuf[e, pl.ds(j * 16, 16)] = rbuf[e, pl.ds(j * 16, 16)] * a

    N_QUAD = -(-(IDX_GRP + DEPTH) // DEPTH)
    for grp in range(NGRP):
        slot = grp % 2
        nslot = (grp + 1) % 2
        if grp > 0:
            pltpu.make_async_copy(
                row_hbm.at[s, pl.ds(grp * IDX_GRP, IDX_GRP)], row_v.at[slot],
                semi.at[slot]).wait()
            pltpu.make_async_copy(
                col_hbm.at[c, s, pl.ds(grp * IDX_GRP, IDX_GRP)], col_v.at[slot],
                semi.at[slot]).wait()
        if grp + 1 < NGRP:
            pltpu.async_copy(
                row_hbm.at[s, pl.ds((grp + 1) * IDX_GRP, IDX_GRP)],
                row_v.at[nslot], semi.at[nslot])
            pltpu.async_copy(
                col_hbm.at[c, s, pl.ds((grp + 1) * IDX_GRP, IDX_GRP)],
                col_v.at[nslot], semi.at[nslot])

        def quad(p, _, slot=slot):
            k0 = p * DEPTH
            for i in range(DEPTH):
                k = k0 + i
                kc = k - LAG
                bc = (i - LAG) % DEPTH

                @pl.when((k >= DEPTH) & (k < IDX_GRP + DEPTH))
                def _():
                    scatter_wait(slot, k - DEPTH, i)

                @pl.when(k < IDX_GRP)
                def _():
                    gather_start(slot, k, i)

                @pl.when((kc >= 0) & (kc < IDX_GRP))
                def _():
                    gather_wait(slot, kc, bc)
                    compute(slot, kc, bc)
                    scatter_start(slot, kc, bc)
            return 0
        lax.fori_loop(0, N_QUAD, quad, 0)

    plsc.subcore_barrier()
    pltpu.sync_copy(acc_sh.at[pl.ds(base, ROWS_PER_TILE)],
                    out_hbm.at[c, pl.ds(base, ROWS_PER_TILE)])
    pltpu.sync_copy(accr_sh.at[pl.ds(base, ROWS_PER_TILE)],
                    outr_hbm.at[c, pl.ds(base, ROWS_PER_TILE)])


_edge_kernel = functools.partial(
    pl.kernel,
    _edge_body,
    out_type=(
        jax.ShapeDtypeStruct((NC, B_ACC, DH), jnp.float32),
        jax.ShapeDtypeStruct((NC, B_ACC, 16), jnp.float32),
    ),
    mesh=plsc.VectorSubcoreMesh(core_axis_name="c", subcore_axis_name="s"),
    compiler_params=pltpu.CompilerParams(
        needs_layout_passes=False, use_tc_tiling_on_sc=False),
    scratch_types=[
        pltpu.VMEM((2, IDX_GRP, CHUNK), jnp.int32),
        pltpu.VMEM((2, IDX_GRP, CHUNK), jnp.int32),
        pltpu.VMEM((B_PAD,), jnp.float32),
        pltpu.VMEM((B_PAD,), jnp.float32),
        pltpu.VMEM((DEPTH, CHUNK, DH), jnp.float32),
        pltpu.VMEM((DEPTH, CHUNK, 16), jnp.float32),
        pltpu.VMEM((CHUNK,), jnp.float32),
        pltpu.VMEM_SHARED((B_ACC, DH), jnp.float32),
        pltpu.VMEM_SHARED((B_ACC, 16), jnp.float32),
        pltpu.SemaphoreType.DMA((DEPTH,)),
        pltpu.SemaphoreType.DMA((DEPTH,)),
        pltpu.SemaphoreType.DMA((DEPTH,)),
        pltpu.SemaphoreType.DMA((2,)),
    ],
)()


def kernel(batch_ids, batch_adj_arr, ent_embed, feature_dropout, W, b, w_atten_r, prelu_a):
    w1 = w_atten_r[:D, 0]
    w2 = w_atten_r[D:, 0]
    W_aug = jnp.stack([W[:, :DH], W[:, DH:]])
    b_aug = jnp.stack([b[None, :DH], b[None, DH:]])
    wv1 = W @ w1
    wv2 = W @ w2
    ent_pad = jnp.zeros((B_PAD, D), jnp.float32).at[:B].set(ent_embed)

    tab, al2, ar2 = pl.pallas_call(
        _prep_body,
        out_shape=(
            jax.ShapeDtypeStruct((NC, B_PAD, DH), jnp.float32),
            jax.ShapeDtypeStruct((B_PAD // R_BLK, 1, R_BLK), jnp.float32),
            jax.ShapeDtypeStruct((B_PAD // R_BLK, 1, R_BLK), jnp.float32),
        ),
        grid=(B_PAD // R_BLK, NC),
        in_specs=[
            pl.BlockSpec((R_BLK, D), lambda i, c: (i, 0)),
            pl.BlockSpec((1, D, DH), lambda i, c: (c, 0, 0)),
            pl.BlockSpec((1, 1, DH), lambda i, c: (c, 0, 0)),
            pl.BlockSpec((1, D), lambda i, c: (0, 0)),
            pl.BlockSpec((1, D), lambda i, c: (0, 0)),
        ],
        out_specs=(
            pl.BlockSpec((1, R_BLK, DH), lambda i, c: (c, i, 0)),
            pl.BlockSpec((1, 1, R_BLK), lambda i, c: (i, 0, 0)),
            pl.BlockSpec((1, 1, R_BLK), lambda i, c: (i, 0, 0)),
        ),
    )(ent_pad, W_aug, b_aug, wv1[None, :], wv2[None, :])
    tab_flat = tab.reshape(NC * B_PAD, DH)
    al = al2.reshape(B_PAD) + jnp.dot(b, w1)
    ar = ar2.reshape(B_PAD) + jnp.dot(b, w2)
    ar = ar.at[B:].set(jnp.float32(1e9))

    row = jnp.concatenate([batch_adj_arr[0], jnp.arange(B, dtype=jnp.int32)])
    col = jnp.concatenate([batch_adj_arr[1], batch_ids.astype(jnp.int32)])
    rowp = jnp.zeros((E_PAD,), jnp.int32).at[:E_TOT].set(row).reshape(NS, CH_PER_S, CHUNK)
    colp = jnp.full((E_PAD,), B, jnp.int32).at[:E_TOT].set(col).reshape(NS, CH_PER_S, CHUNK)
    colp2 = jnp.stack([colp, colp + B_PAD])

    parts, rsums = _edge_kernel(tab_flat, al, ar, rowp, colp2)

    return pl.pallas_call(
        _fin_body,
        out_shape=jax.ShapeDtypeStruct((B, D), jnp.float32),
        grid=(B // F_BLK,),
        in_specs=[
            pl.BlockSpec((NC, F_BLK, DH), lambda i: (0, i, 0)),
            pl.BlockSpec((NC, F_BLK, 16), lambda i: (0, i, 0)),
            pl.BlockSpec((1, 1), lambda i: (0, 0)),
        ],
        out_specs=pl.BlockSpec((F_BLK, D), lambda i: (i, 0)),
    )(parts, rsums, jnp.float32(prelu_a).reshape(1, 1))
